# baseline (speedup 1.0000x reference)
import functools

import jax
import jax.numpy as jnp
from jax import lax
from jax.experimental import pallas as pl
from jax.experimental.pallas import tpu as pltpu
from jax.experimental.pallas import tpu_sc as plsc

_NBUF = 8

_info = plsc.get_sparse_core_info()
_NC = _info.num_cores
_NS = _info.num_subcores
_NW = _NC * _NS


@functools.partial(jax.jit, static_argnames=("dim",))
def _gather(idx, table, *, dim):
    n_b, n_s = idx.shape
    b_per_w = n_b // _NW
    mesh = plsc.VectorSubcoreMesh(core_axis_name="c", subcore_axis_name="s")

    @functools.partial(
        pl.kernel,
        out_type=jax.ShapeDtypeStruct((n_b, n_s, dim), jnp.float32),
        mesh=mesh,
        scratch_types=[
            pltpu.VMEM((b_per_w, n_s), jnp.int32),
            pltpu.VMEM((_NBUF, n_s, dim), jnp.float32),
        ]
        + [pltpu.SemaphoreType.DMA] * _NBUF,
        compiler_params=pltpu.CompilerParams(use_tc_tiling_on_sc=False),
    )
    def k(idx_hbm, table_hbm, out_hbm, idx_v, rows_v, *sems):
        wid = lax.axis_index("s") * _NC + lax.axis_index("c")
        b0 = wid * b_per_w
        pltpu.sync_copy(idx_hbm.at[pl.ds(b0, b_per_w)], idx_v)

        def fire(buf, j):
            pltpu.async_copy(table_hbm.at[idx_v.at[j]], rows_v.at[buf], sems[buf])

        def drain(buf):
            pltpu.make_async_copy(
                table_hbm.at[idx_v.at[0]], rows_v.at[buf], sems[buf]
            ).wait()

        for b in range(_NBUF):
            fire(b, b)

        @pl.loop(0, b_per_w, step=_NBUF)
        def row_body(j0):
            for b in range(_NBUF):
                j = j0 + b
                drain(b)
                pltpu.sync_copy(rows_v.at[b], out_hbm.at[b0 + j])

                @pl.when(j + _NBUF < b_per_w)
                def _():
                    fire(b, j + _NBUF)

    return k(idx, table)


def kernel(indices, embeddings):
    dim = embeddings.shape[1]
    return _gather(indices, embeddings, dim=dim)


# speedup vs baseline: 1.0000x; 1.0000x over previous
import functools

import jax
import jax.experimental.layout
import jax.numpy as jnp
from jax import lax
from jax.experimental import pallas as pl
from jax.experimental.pallas import tpu as pltpu
from jax.experimental.pallas import tpu_sc as plsc

_NBUF = 8

_info = plsc.get_sparse_core_info()
_NC = _info.num_cores
_NS = _info.num_subcores
_NW = _NC * _NS


@functools.partial(jax.jit, static_argnames=("dim",))
def _gather(idx, table, *, dim):
    n_b, n_s = idx.shape
    b_per_w = n_b // _NW
    mesh = plsc.VectorSubcoreMesh(core_axis_name="c", subcore_axis_name="s")

    @functools.partial(
        pl.kernel,
        out_type=jax.ShapeDtypeStruct((n_b, n_s, dim), jnp.float32),
        mesh=mesh,
        scratch_types=[
            pltpu.VMEM((b_per_w, n_s), jnp.int32),
            pltpu.VMEM((_NBUF, n_s, dim), jnp.float32),
        ]
        + [pltpu.SemaphoreType.DMA] * _NBUF,
        compiler_params=pltpu.CompilerParams(use_tc_tiling_on_sc=False),
    )
    def k(idx_hbm, table_hbm, out_hbm, idx_v, rows_v, *sems):
        wid = lax.axis_index("s") * _NC + lax.axis_index("c")
        b0 = wid * b_per_w
        pltpu.sync_copy(idx_hbm.at[pl.ds(b0, b_per_w)], idx_v)

        def fire(buf, j):
            pltpu.async_copy(table_hbm.at[idx_v.at[j]], rows_v.at[buf], sems[buf])

        def drain(buf):
            pltpu.make_async_copy(
                table_hbm.at[idx_v.at[0]], rows_v.at[buf], sems[buf]
            ).wait()

        for b in range(_NBUF):
            fire(b, b)

        @pl.loop(0, b_per_w, step=_NBUF)
        def row_body(j0):
            for b in range(_NBUF):
                j = j0 + b
                drain(b)
                pltpu.sync_copy(rows_v.at[b], out_hbm.at[b0 + j])

                @pl.when(j + _NBUF < b_per_w)
                def _():
                    fire(b, j + _NBUF)

    return k(idx, table)


def _jitted_kernel():
    fmt = jax.experimental.layout.Format(
        jax.experimental.layout.Layout(major_to_minor=(0, 1, 2)),
        jax.sharding.SingleDeviceSharding(jax.devices()[0]),
    )

    @functools.partial(jax.jit, out_shardings=fmt)
    def run(indices, embeddings):
        return _gather(indices, embeddings, dim=embeddings.shape[1])

    return run


def kernel(indices, embeddings):
    return _jitted_kernel()(indices, embeddings)


# speedup vs baseline: 1.0004x; 1.0004x over previous
import functools

import jax
import jax.numpy as jnp
from jax import lax
from jax.experimental import pallas as pl
from jax.experimental.pallas import tpu as pltpu
from jax.experimental.pallas import tpu_sc as plsc

_NBUF = 8

_info = plsc.get_sparse_core_info()
_NC = _info.num_cores
_NS = _info.num_subcores
_NW = _NC * _NS


@functools.partial(jax.jit, static_argnames=("dim",))
def _gather(idx, table, *, dim):
    n_b, n_s = idx.shape
    b_per_w = n_b // _NW
    mesh = plsc.VectorSubcoreMesh(core_axis_name="c", subcore_axis_name="s")

    @functools.partial(
        pl.kernel,
        out_type=jax.ShapeDtypeStruct((n_b, n_s, dim), jnp.float32),
        mesh=mesh,
        scratch_types=[
            pltpu.VMEM((b_per_w, n_s), jnp.int32),
            pltpu.VMEM((_NBUF, n_s, dim), jnp.float32),
        ]
        + [pltpu.SemaphoreType.DMA] * _NBUF,
        compiler_params=pltpu.CompilerParams(use_tc_tiling_on_sc=False),
    )
    def k(idx_hbm, table_hbm, out_hbm, idx_v, rows_v, *sems):
        wid = lax.axis_index("s") * _NC + lax.axis_index("c")
        b0 = wid * b_per_w
        pltpu.sync_copy(idx_hbm.at[pl.ds(b0, b_per_w)], idx_v)

        def fire(buf, j):
            pltpu.async_copy(table_hbm.at[idx_v.at[j]], rows_v.at[buf], sems[buf])

        def drain(buf):
            pltpu.make_async_copy(
                table_hbm.at[idx_v.at[0]], rows_v.at[buf], sems[buf]
            ).wait()

        for b in range(_NBUF):
            fire(b, b)

        @pl.loop(0, b_per_w, step=_NBUF)
        def row_body(j0):
            for b in range(_NBUF):
                j = j0 + b
                drain(b)
                pltpu.sync_copy(rows_v.at[b], out_hbm.at[b0 + j])

                @pl.when(j + _NBUF < b_per_w)
                def _():
                    fire(b, j + _NBUF)

    return k(idx, table)


def kernel(indices, embeddings):
    return _gather(indices, embeddings, dim=embeddings.shape[1])
